# Initial kernel scaffold; baseline (speedup 1.0000x reference)
import jax
import jax.numpy as jnp
from jax.experimental import pallas as pl


def kernel(x, W_down, b_down, W_in, b_in, codebook, W_out, b_out, W_up, b_up):
    raise NotImplementedError("write your pallas kernel here")


# speedup vs baseline: 3.3301x; 3.3301x over previous
import jax
import jax.numpy as jnp
from jax.experimental import pallas as pl
from jax.experimental.pallas import tpu as pltpu

B = 65536
IN_FEAT = 128
HIDDEN = 128
OUT_FEAT = 128
CODEBOOK_DIM = 32
NUM_CODES = 256

BLOCK_B = 2048


def _fold_kernel(cb_ref, wo_ref, bo_ref, wu_ref, bu_ref, table_ref):
    t = jnp.dot(cb_ref[...], wo_ref[...],
                preferred_element_type=jnp.float32) + bo_ref[...]
    t = jnp.dot(t, wu_ref[...],
                preferred_element_type=jnp.float32) + bu_ref[...]
    table_ref[...] = jnp.clip(t, -1.0, 1.0)


def _main_kernel(x_ref, wd_ref, bd_ref, wi_ref, bi_ref, cb_ref, table_ref,
                 out_ref, idx_ref):
    h = jnp.dot(x_ref[...], wd_ref[...],
                preferred_element_type=jnp.float32) + bd_ref[...]
    z = jnp.dot(h, wi_ref[...],
                preferred_element_type=jnp.float32) + bi_ref[...]
    cb = cb_ref[...]
    d2 = (jnp.sum(z * z, axis=-1, keepdims=True)
          - 2.0 * jnp.dot(z, cb.T, preferred_element_type=jnp.float32)
          + jnp.sum(cb * cb, axis=-1)[None, :])
    idx = jnp.argmin(d2, axis=-1).astype(jnp.int32)
    idx_ref[...] = idx[:, None]
    onehot = (jax.lax.broadcasted_iota(jnp.int32, (BLOCK_B, NUM_CODES), 1)
              == idx[:, None]).astype(jnp.float32)
    out_ref[...] = jnp.dot(onehot, table_ref[...],
                           preferred_element_type=jnp.float32)


def kernel(x, W_down, b_down, W_in, b_in, codebook, W_out, b_out, W_up, b_up):
    table = pl.pallas_call(
        _fold_kernel,
        out_shape=jax.ShapeDtypeStruct((NUM_CODES, OUT_FEAT), jnp.float32),
    )(codebook, W_out, b_out, W_up, b_up)

    grid = (B // BLOCK_B,)
    out, idx2d = pl.pallas_call(
        _main_kernel,
        grid=grid,
        in_specs=[
            pl.BlockSpec((BLOCK_B, IN_FEAT), lambda i: (i, 0)),
            pl.BlockSpec((IN_FEAT, HIDDEN), lambda i: (0, 0)),
            pl.BlockSpec((HIDDEN,), lambda i: (0,)),
            pl.BlockSpec((HIDDEN, CODEBOOK_DIM), lambda i: (0, 0)),
            pl.BlockSpec((CODEBOOK_DIM,), lambda i: (0,)),
            pl.BlockSpec((NUM_CODES, CODEBOOK_DIM), lambda i: (0, 0)),
            pl.BlockSpec((NUM_CODES, OUT_FEAT), lambda i: (0, 0)),
        ],
        out_specs=(
            pl.BlockSpec((BLOCK_B, OUT_FEAT), lambda i: (i, 0)),
            pl.BlockSpec((BLOCK_B, 1), lambda i: (i, 0)),
        ),
        out_shape=(
            jax.ShapeDtypeStruct((B, OUT_FEAT), jnp.float32),
            jax.ShapeDtypeStruct((B, 1), jnp.int32),
        ),
        compiler_params=pltpu.CompilerParams(
            dimension_semantics=("arbitrary",),
        ),
    )(x, W_down, b_down, W_in, b_in, codebook, table)

    indices = idx2d.reshape(B)
    commit_loss = jnp.zeros((), jnp.float32)
    return out, indices, commit_loss


# speedup vs baseline: 3.6513x; 1.0965x over previous
import jax
import jax.numpy as jnp
from jax.experimental import pallas as pl
from jax.experimental.pallas import tpu as pltpu

B = 65536
IN_FEAT = 128
HIDDEN = 128
OUT_FEAT = 128
CODEBOOK_DIM = 32
NUM_CODES = 256

BLOCK_B = 2048


def _fold_kernel(cb_ref, wo_ref, bo_ref, wu_ref, bu_ref, table_ref):
    t = jnp.dot(cb_ref[...], wo_ref[...],
                preferred_element_type=jnp.float32) + bo_ref[...]
    t = jnp.dot(t, wu_ref[...],
                preferred_element_type=jnp.float32) + bu_ref[...]
    table_ref[...] = jnp.clip(t, -1.0, 1.0)


def _main_kernel(x_ref, wd_ref, bd_ref, wi_ref, bi_ref, cb_ref, table_ref,
                 out_ref, idx_ref):
    h = jnp.dot(x_ref[...], wd_ref[...],
                preferred_element_type=jnp.float32) + bd_ref[...]
    z = jnp.dot(h, wi_ref[...],
                preferred_element_type=jnp.float32) + bi_ref[...]
    cb = cb_ref[...]
    zc = jnp.dot(z, cb.T, preferred_element_type=jnp.float32)
    s = zc - 0.5 * jnp.sum(cb * cb, axis=-1)[None, :]
    idx = jnp.argmax(s, axis=-1).astype(jnp.int32)
    idx_ref[...] = idx[:, None]
    onehot = (jax.lax.broadcasted_iota(jnp.int32, (BLOCK_B, NUM_CODES), 1)
              == idx[:, None]).astype(jnp.float32)
    out_ref[...] = jnp.dot(onehot, table_ref[...],
                           preferred_element_type=jnp.float32)


def kernel(x, W_down, b_down, W_in, b_in, codebook, W_out, b_out, W_up, b_up):
    table = pl.pallas_call(
        _fold_kernel,
        out_shape=jax.ShapeDtypeStruct((NUM_CODES, OUT_FEAT), jnp.float32),
    )(codebook, W_out, b_out, W_up, b_up)

    grid = (B // BLOCK_B,)
    out, idx2d = pl.pallas_call(
        _main_kernel,
        grid=grid,
        in_specs=[
            pl.BlockSpec((BLOCK_B, IN_FEAT), lambda i: (i, 0)),
            pl.BlockSpec((IN_FEAT, HIDDEN), lambda i: (0, 0)),
            pl.BlockSpec((HIDDEN,), lambda i: (0,)),
            pl.BlockSpec((HIDDEN, CODEBOOK_DIM), lambda i: (0, 0)),
            pl.BlockSpec((CODEBOOK_DIM,), lambda i: (0,)),
            pl.BlockSpec((NUM_CODES, CODEBOOK_DIM), lambda i: (0, 0)),
            pl.BlockSpec((NUM_CODES, OUT_FEAT), lambda i: (0, 0)),
        ],
        out_specs=(
            pl.BlockSpec((BLOCK_B, OUT_FEAT), lambda i: (i, 0)),
            pl.BlockSpec((BLOCK_B, 1), lambda i: (i, 0)),
        ),
        out_shape=(
            jax.ShapeDtypeStruct((B, OUT_FEAT), jnp.float32),
            jax.ShapeDtypeStruct((B, 1), jnp.int32),
        ),
        compiler_params=pltpu.CompilerParams(
            dimension_semantics=("parallel",),
        ),
    )(x, W_down, b_down, W_in, b_in, codebook, table)

    indices = idx2d.reshape(B)
    commit_loss = jnp.zeros((), jnp.float32)
    return out, indices, commit_loss


# speedup vs baseline: 4.0960x; 1.1218x over previous
import jax
import jax.numpy as jnp
from jax.experimental import pallas as pl
from jax.experimental.pallas import tpu as pltpu

B = 65536
IN_FEAT = 128
HIDDEN = 128
OUT_FEAT = 128
CODEBOOK_DIM = 32
NUM_CODES = 256

BLOCK_B = 4096


def _fold_kernel(cb_ref, wo_ref, bo_ref, wu_ref, bu_ref, table_ref):
    t = jnp.dot(cb_ref[...], wo_ref[...],
                preferred_element_type=jnp.float32) + bo_ref[...]
    t = jnp.dot(t, wu_ref[...],
                preferred_element_type=jnp.float32) + bu_ref[...]
    table_ref[...] = jnp.clip(t, -1.0, 1.0)


def _main_kernel(x_ref, wd_ref, bd_ref, wi_ref, bi_ref, cb_ref, table_ref,
                 out_ref, idx_ref):
    h = jnp.dot(x_ref[...], wd_ref[...],
                preferred_element_type=jnp.float32) + bd_ref[...]
    z = jnp.dot(h, wi_ref[...],
                preferred_element_type=jnp.float32) + bi_ref[...]
    cb = cb_ref[...]
    zc = jnp.dot(z, cb.T, preferred_element_type=jnp.float32)
    s = zc - 0.5 * jnp.sum(cb * cb, axis=-1)[None, :]
    idx = jnp.argmax(s, axis=-1).astype(jnp.int32)
    idx_ref[...] = idx[:, None]
    onehot = (jax.lax.broadcasted_iota(jnp.int32, (BLOCK_B, NUM_CODES), 1)
              == idx[:, None]).astype(jnp.float32)
    out_ref[...] = jnp.dot(onehot, table_ref[...],
                           preferred_element_type=jnp.float32)


def kernel(x, W_down, b_down, W_in, b_in, codebook, W_out, b_out, W_up, b_up):
    table = pl.pallas_call(
        _fold_kernel,
        out_shape=jax.ShapeDtypeStruct((NUM_CODES, OUT_FEAT), jnp.float32),
    )(codebook, W_out, b_out, W_up, b_up)

    grid = (B // BLOCK_B,)
    out, idx2d = pl.pallas_call(
        _main_kernel,
        grid=grid,
        in_specs=[
            pl.BlockSpec((BLOCK_B, IN_FEAT), lambda i: (i, 0)),
            pl.BlockSpec((IN_FEAT, HIDDEN), lambda i: (0, 0)),
            pl.BlockSpec((HIDDEN,), lambda i: (0,)),
            pl.BlockSpec((HIDDEN, CODEBOOK_DIM), lambda i: (0, 0)),
            pl.BlockSpec((CODEBOOK_DIM,), lambda i: (0,)),
            pl.BlockSpec((NUM_CODES, CODEBOOK_DIM), lambda i: (0, 0)),
            pl.BlockSpec((NUM_CODES, OUT_FEAT), lambda i: (0, 0)),
        ],
        out_specs=(
            pl.BlockSpec((BLOCK_B, OUT_FEAT), lambda i: (i, 0)),
            pl.BlockSpec((BLOCK_B, 1), lambda i: (i, 0)),
        ),
        out_shape=(
            jax.ShapeDtypeStruct((B, OUT_FEAT), jnp.float32),
            jax.ShapeDtypeStruct((B, 1), jnp.int32),
        ),
        compiler_params=pltpu.CompilerParams(
            dimension_semantics=("parallel",),
        ),
    )(x, W_down, b_down, W_in, b_in, codebook, table)

    indices = idx2d.reshape(B)
    commit_loss = jnp.zeros((), jnp.float32)
    return out, indices, commit_loss


# speedup vs baseline: 4.2235x; 1.0311x over previous
import jax
import jax.numpy as jnp
from jax.experimental import pallas as pl
from jax.experimental.pallas import tpu as pltpu

B = 65536
IN_FEAT = 128
HIDDEN = 128
OUT_FEAT = 128
CODEBOOK_DIM = 32
NUM_CODES = 256

BLOCK_B = 8192


def _fold_kernel(cb_ref, wo_ref, bo_ref, wu_ref, bu_ref, table_ref):
    t = jnp.dot(cb_ref[...], wo_ref[...],
                preferred_element_type=jnp.float32) + bo_ref[...]
    t = jnp.dot(t, wu_ref[...],
                preferred_element_type=jnp.float32) + bu_ref[...]
    table_ref[...] = jnp.clip(t, -1.0, 1.0)


def _main_kernel(x_ref, wd_ref, bd_ref, wi_ref, bi_ref, cb_ref, table_ref,
                 out_ref, idx_ref):
    h = jnp.dot(x_ref[...], wd_ref[...],
                preferred_element_type=jnp.float32) + bd_ref[...]
    z = jnp.dot(h, wi_ref[...],
                preferred_element_type=jnp.float32) + bi_ref[...]
    cb = cb_ref[...]
    zc = jnp.dot(z, cb.T, preferred_element_type=jnp.float32)
    s = zc - 0.5 * jnp.sum(cb * cb, axis=-1)[None, :]
    idx = jnp.argmax(s, axis=-1).astype(jnp.int32)
    idx_ref[...] = idx[:, None]
    onehot = (jax.lax.broadcasted_iota(jnp.int32, (BLOCK_B, NUM_CODES), 1)
              == idx[:, None]).astype(jnp.float32)
    out_ref[...] = jnp.dot(onehot, table_ref[...],
                           preferred_element_type=jnp.float32)


def kernel(x, W_down, b_down, W_in, b_in, codebook, W_out, b_out, W_up, b_up):
    table = pl.pallas_call(
        _fold_kernel,
        out_shape=jax.ShapeDtypeStruct((NUM_CODES, OUT_FEAT), jnp.float32),
    )(codebook, W_out, b_out, W_up, b_up)

    grid = (B // BLOCK_B,)
    out, idx2d = pl.pallas_call(
        _main_kernel,
        grid=grid,
        in_specs=[
            pl.BlockSpec((BLOCK_B, IN_FEAT), lambda i: (i, 0)),
            pl.BlockSpec((IN_FEAT, HIDDEN), lambda i: (0, 0)),
            pl.BlockSpec((HIDDEN,), lambda i: (0,)),
            pl.BlockSpec((HIDDEN, CODEBOOK_DIM), lambda i: (0, 0)),
            pl.BlockSpec((CODEBOOK_DIM,), lambda i: (0,)),
            pl.BlockSpec((NUM_CODES, CODEBOOK_DIM), lambda i: (0, 0)),
            pl.BlockSpec((NUM_CODES, OUT_FEAT), lambda i: (0, 0)),
        ],
        out_specs=(
            pl.BlockSpec((BLOCK_B, OUT_FEAT), lambda i: (i, 0)),
            pl.BlockSpec((BLOCK_B, 1), lambda i: (i, 0)),
        ),
        out_shape=(
            jax.ShapeDtypeStruct((B, OUT_FEAT), jnp.float32),
            jax.ShapeDtypeStruct((B, 1), jnp.int32),
        ),
        compiler_params=pltpu.CompilerParams(
            dimension_semantics=("parallel",),
        ),
    )(x, W_down, b_down, W_in, b_in, codebook, table)

    indices = idx2d.reshape(B)
    commit_loss = jnp.zeros((), jnp.float32)
    return out, indices, commit_loss
